# baseline (speedup 1.0000x reference)
import functools

import jax
import jax.numpy as jnp
from jax import lax
from jax.experimental import pallas as pl
from jax.experimental.pallas import tpu as pltpu
from jax.experimental.pallas import tpu_sc as plsc

N_USERS = 1000000
N_ITEMS = 1000000
N_FACTORS = 128
BATCH = 16384

NC = 2
NS = 16
NW = NC * NS
B_PER_W = BATCH // NW
CHUNK = 128
NCHUNK = B_PER_W // CHUNK
TAIL0 = 999424
TAIL_N = N_USERS - TAIL0


def _shuf(x, idx):
    return lax.gather(
        x, idx[:, None],
        dimension_numbers=lax.GatherDimensionNumbers(
            offset_dims=(), collapsed_slice_dims=(0,), start_index_map=(0,)),
        slice_sizes=(1,),
        mode=lax.GatherScatterMode.PROMISE_IN_BOUNDS,
    )


def _dots_body(users_hbm, items_hbm, uf_hbm, if_hbm, out_hbm,
               u_idx, i_idx, uf_a, if_a, uf_b, if_b, p_dot, out_v,
               sem_a, sem_b):
    wid = lax.axis_index("s") * NC + lax.axis_index("c")
    base = wid * B_PER_W
    pltpu.sync_copy(users_hbm.at[pl.ds(base, B_PER_W)], u_idx)
    pltpu.sync_copy(items_hbm.at[pl.ds(base, B_PER_W)], i_idx)
    lane = lax.iota(jnp.int32, 16)

    bufs = [(uf_a, if_a), (uf_b, if_b)]
    sems = [sem_a, sem_b]

    def fire(c):
        sl = pl.ds(c * CHUNK, CHUNK)
        uf_r, if_r = bufs[c & 1]
        sem = sems[c & 1]
        return [
            pltpu.async_copy(uf_hbm.at[u_idx.at[sl]], uf_r, sem),
            pltpu.async_copy(if_hbm.at[i_idx.at[sl]], if_r, sem),
        ]

    pending = fire(0)
    for c in range(NCHUNK):
        nxt = fire(c + 1) if c + 1 < NCHUNK else None
        for cp in pending:
            cp.wait()
        pending = nxt
        uf_rows, if_rows = bufs[c & 1]

        @plsc.parallel_loop(0, CHUNK, unroll=4)
        def rowbody(r):
            t0 = uf_rows[r, pl.ds(0, 16)] * if_rows[r, pl.ds(0, 16)]
            t1 = uf_rows[r, pl.ds(16, 16)] * if_rows[r, pl.ds(16, 16)]
            t2 = uf_rows[r, pl.ds(32, 16)] * if_rows[r, pl.ds(32, 16)]
            t3 = uf_rows[r, pl.ds(48, 16)] * if_rows[r, pl.ds(48, 16)]
            t4 = uf_rows[r, pl.ds(64, 16)] * if_rows[r, pl.ds(64, 16)]
            t5 = uf_rows[r, pl.ds(80, 16)] * if_rows[r, pl.ds(80, 16)]
            t6 = uf_rows[r, pl.ds(96, 16)] * if_rows[r, pl.ds(96, 16)]
            t7 = uf_rows[r, pl.ds(112, 16)] * if_rows[r, pl.ds(112, 16)]
            s = ((t0 + t1) + (t2 + t3)) + ((t4 + t5) + (t6 + t7))
            for k in (8, 4, 2, 1):
                s = s + _shuf(s, lane ^ k)
            p_dot[r, pl.ds(0, 16)] = s

        def gout(g, _):
            off = c * CHUNK + g * 16
            d16 = plsc.load_gather(p_dot, [g * 16 + lane, lane])
            out_v[pl.ds(off, 16)] = d16
            return 0

        lax.fori_loop(0, CHUNK // 16, gout, 0)
    pltpu.sync_copy(out_v, out_hbm.at[pl.ds(base, B_PER_W)])


def _bias_body(users_hbm, items_hbm, ub_hbm, ib_hbm,
               ubt_hbm, ibt_hbm, out_hbm,
               u_idx, i_idx, cu_idx, ci_idx, ub_v, ib_v, ub_tl, ib_tl,
               out_v, sem):
    wid = lax.axis_index("s") * NC + lax.axis_index("c")
    base = wid * B_PER_W
    pltpu.sync_copy(users_hbm.at[pl.ds(base, B_PER_W)], u_idx)
    pltpu.sync_copy(items_hbm.at[pl.ds(base, B_PER_W)], i_idx)
    pltpu.sync_copy(ubt_hbm, ub_tl)
    pltpu.sync_copy(ibt_hbm, ib_tl)
    lane = lax.iota(jnp.int32, 16)

    def crow(g, _):
        gsl = pl.ds(g * 16, 16)
        cu_idx[gsl] = jnp.minimum(u_idx[gsl], TAIL0 - 1)
        ci_idx[gsl] = jnp.minimum(i_idx[gsl], TAIL0 - 1)
        return 0

    lax.fori_loop(0, B_PER_W // 16, crow, 0)

    cps = []
    for c in range(NCHUNK):
        sl = pl.ds(c * CHUNK, CHUNK)
        cps.append(pltpu.async_copy(ub_hbm.at[cu_idx.at[sl]], ub_v.at[sl], sem))
        cps.append(pltpu.async_copy(ib_hbm.at[ci_idx.at[sl]], ib_v.at[sl], sem))
    for cp in cps:
        cp.wait()

    def gout(g, _):
        gsl = pl.ds(g * 16, 16)
        u16 = u_idx[gsl]
        i16 = i_idx[gsl]
        ub16 = ub_v[gsl]
        ib16 = ib_v[gsl]
        ubt16 = plsc.load_gather(
            ub_tl, [jnp.maximum(jnp.minimum(u16 - TAIL0, TAIL_N - 1), 0)])
        ibt16 = plsc.load_gather(
            ib_tl, [jnp.maximum(jnp.minimum(i16 - TAIL0, TAIL_N - 1), 0)])
        ub16 = jnp.where(u16 >= TAIL0, ubt16, ub16)
        ib16 = jnp.where(i16 >= TAIL0, ibt16, ib16)
        out_v[gsl] = ub16 + ib16
        return 0

    lax.fori_loop(0, B_PER_W // 16, gout, 0)
    pltpu.sync_copy(out_v, out_hbm.at[pl.ds(base, B_PER_W)])


@jax.jit
def kernel(users, items, user_factors, item_factors, user_biases, item_biases):
    ub_flat = user_biases[:TAIL0].reshape(TAIL0)
    ib_flat = item_biases[:TAIL0].reshape(TAIL0)
    ub_tail = user_biases[TAIL0:].reshape(TAIL_N)
    ib_tail = item_biases[TAIL0:].reshape(TAIL_N)

    mesh = plsc.VectorSubcoreMesh(core_axis_name="c", subcore_axis_name="s")
    run_dots = functools.partial(
        pl.kernel,
        out_type=jax.ShapeDtypeStruct((BATCH,), jnp.float32),
        mesh=mesh,
        compiler_params=pltpu.CompilerParams(needs_layout_passes=False),
        scratch_types=[
            pltpu.VMEM((B_PER_W,), jnp.int32),
            pltpu.VMEM((B_PER_W,), jnp.int32),
            pltpu.VMEM((CHUNK, N_FACTORS), jnp.float32),
            pltpu.VMEM((CHUNK, N_FACTORS), jnp.float32),
            pltpu.VMEM((CHUNK, N_FACTORS), jnp.float32),
            pltpu.VMEM((CHUNK, N_FACTORS), jnp.float32),
            pltpu.VMEM((CHUNK, 16), jnp.float32),
            pltpu.VMEM((B_PER_W,), jnp.float32),
            pltpu.SemaphoreType.DMA,
            pltpu.SemaphoreType.DMA,
        ],
    )(_dots_body)
    dots = run_dots(users, items, user_factors, item_factors)

    run_bias = functools.partial(
        pl.kernel,
        out_type=jax.ShapeDtypeStruct((BATCH,), jnp.float32),
        mesh=mesh,
        compiler_params=pltpu.CompilerParams(needs_layout_passes=False),
        scratch_types=[
            pltpu.VMEM((B_PER_W,), jnp.int32),
            pltpu.VMEM((B_PER_W,), jnp.int32),
            pltpu.VMEM((B_PER_W,), jnp.int32),
            pltpu.VMEM((B_PER_W,), jnp.int32),
            pltpu.VMEM((B_PER_W,), jnp.float32),
            pltpu.VMEM((B_PER_W,), jnp.float32),
            pltpu.VMEM((TAIL_N,), jnp.float32),
            pltpu.VMEM((TAIL_N,), jnp.float32),
            pltpu.VMEM((B_PER_W,), jnp.float32),
            pltpu.SemaphoreType.DMA,
        ],
    )(_bias_body)
    bias_sum = run_bias(users, items, ub_flat, ib_flat, ub_tail, ib_tail)
    return dots + bias_sum


# speedup vs baseline: 1.2301x; 1.2301x over previous
import functools

import jax
import jax.numpy as jnp
from jax import lax
from jax.experimental import pallas as pl
from jax.experimental.pallas import tpu as pltpu
from jax.experimental.pallas import tpu_sc as plsc

N_USERS = 1000000
N_ITEMS = 1000000
N_FACTORS = 128
BATCH = 16384

NC = 2
NS = 16
NW = NC * NS
B_PER_W = BATCH // NW
CHUNK = 128
NCHUNK = B_PER_W // CHUNK
TAIL0 = 999424
TAIL_N = N_USERS - TAIL0
DCHUNK = 64
DNCHUNK = B_PER_W // DCHUNK


def _shuf(x, idx):
    return lax.gather(
        x, idx[:, None],
        dimension_numbers=lax.GatherDimensionNumbers(
            offset_dims=(), collapsed_slice_dims=(0,), start_index_map=(0,)),
        slice_sizes=(1,),
        mode=lax.GatherScatterMode.PROMISE_IN_BOUNDS,
    )


def _dots_body(users_hbm, items_hbm, uf_hbm, if_hbm, out_hbm,
               u_idx, i_idx, uf_a, if_a, uf_b, if_b, uf_c, if_c, uf_d, if_d,
               p_dot, out_v, sem_a, sem_b, sem_c, sem_d):
    wid = lax.axis_index("s") * NC + lax.axis_index("c")
    base = wid * B_PER_W
    pltpu.sync_copy(users_hbm.at[pl.ds(base, B_PER_W)], u_idx)
    pltpu.sync_copy(items_hbm.at[pl.ds(base, B_PER_W)], i_idx)
    lane = lax.iota(jnp.int32, 16)

    bufs = [(uf_a, if_a), (uf_b, if_b), (uf_c, if_c), (uf_d, if_d)]
    sems = [sem_a, sem_b, sem_c, sem_d]
    NBUF = 4

    def fire(c):
        sl = pl.ds(c * DCHUNK, DCHUNK)
        uf_r, if_r = bufs[c % NBUF]
        sem = sems[c % NBUF]
        return [
            pltpu.async_copy(uf_hbm.at[u_idx.at[sl]], uf_r, sem),
            pltpu.async_copy(if_hbm.at[i_idx.at[sl]], if_r, sem),
        ]

    pending = {c: fire(c) for c in range(min(NBUF - 1, DNCHUNK))}
    for c in range(DNCHUNK):
        if c + NBUF - 1 < DNCHUNK:
            pending[c + NBUF - 1] = fire(c + NBUF - 1)
        for cp in pending.pop(c):
            cp.wait()
        uf_rows, if_rows = bufs[c % NBUF]

        @plsc.parallel_loop(0, DCHUNK, unroll=4)
        def rowbody(r):
            t0 = uf_rows[r, pl.ds(0, 16)] * if_rows[r, pl.ds(0, 16)]
            t1 = uf_rows[r, pl.ds(16, 16)] * if_rows[r, pl.ds(16, 16)]
            t2 = uf_rows[r, pl.ds(32, 16)] * if_rows[r, pl.ds(32, 16)]
            t3 = uf_rows[r, pl.ds(48, 16)] * if_rows[r, pl.ds(48, 16)]
            t4 = uf_rows[r, pl.ds(64, 16)] * if_rows[r, pl.ds(64, 16)]
            t5 = uf_rows[r, pl.ds(80, 16)] * if_rows[r, pl.ds(80, 16)]
            t6 = uf_rows[r, pl.ds(96, 16)] * if_rows[r, pl.ds(96, 16)]
            t7 = uf_rows[r, pl.ds(112, 16)] * if_rows[r, pl.ds(112, 16)]
            s = ((t0 + t1) + (t2 + t3)) + ((t4 + t5) + (t6 + t7))
            for k in (8, 4, 2, 1):
                s = s + _shuf(s, lane ^ k)
            p_dot[r, pl.ds(0, 16)] = s

        def gout(g, _):
            off = c * DCHUNK + g * 16
            d16 = plsc.load_gather(p_dot, [g * 16 + lane, lane])
            out_v[pl.ds(off, 16)] = d16
            return 0

        lax.fori_loop(0, DCHUNK // 16, gout, 0)
    pltpu.sync_copy(out_v, out_hbm.at[pl.ds(base, B_PER_W)])


def _bias_body(users_hbm, items_hbm, dots_hbm, ub_hbm, ib_hbm,
               ubt_hbm, ibt_hbm, out_hbm,
               u_idx, i_idx, cu_idx, ci_idx, ub_v, ib_v, ub_tl, ib_tl,
               dots_v, out_v, sem):
    wid = lax.axis_index("s") * NC + lax.axis_index("c")
    base = wid * B_PER_W
    pltpu.sync_copy(users_hbm.at[pl.ds(base, B_PER_W)], u_idx)
    pltpu.sync_copy(items_hbm.at[pl.ds(base, B_PER_W)], i_idx)
    pltpu.sync_copy(dots_hbm.at[pl.ds(base, B_PER_W)], dots_v)
    pltpu.sync_copy(ubt_hbm, ub_tl)
    pltpu.sync_copy(ibt_hbm, ib_tl)
    lane = lax.iota(jnp.int32, 16)

    def crow(g, _):
        gsl = pl.ds(g * 16, 16)
        cu_idx[gsl] = jnp.minimum(u_idx[gsl], TAIL0 - 1)
        ci_idx[gsl] = jnp.minimum(i_idx[gsl], TAIL0 - 1)
        return 0

    lax.fori_loop(0, B_PER_W // 16, crow, 0)

    cps = []
    for c in range(NCHUNK):
        sl = pl.ds(c * CHUNK, CHUNK)
        cps.append(pltpu.async_copy(ub_hbm.at[cu_idx.at[sl]], ub_v.at[sl], sem))
        cps.append(pltpu.async_copy(ib_hbm.at[ci_idx.at[sl]], ib_v.at[sl], sem))
    for cp in cps:
        cp.wait()

    def gout(g, _):
        gsl = pl.ds(g * 16, 16)
        u16 = u_idx[gsl]
        i16 = i_idx[gsl]
        ub16 = ub_v[gsl]
        ib16 = ib_v[gsl]
        ubt16 = plsc.load_gather(
            ub_tl, [jnp.maximum(jnp.minimum(u16 - TAIL0, TAIL_N - 1), 0)])
        ibt16 = plsc.load_gather(
            ib_tl, [jnp.maximum(jnp.minimum(i16 - TAIL0, TAIL_N - 1), 0)])
        ub16 = jnp.where(u16 >= TAIL0, ubt16, ub16)
        ib16 = jnp.where(i16 >= TAIL0, ibt16, ib16)
        out_v[gsl] = dots_v[gsl] + ub16 + ib16
        return 0

    lax.fori_loop(0, B_PER_W // 16, gout, 0)
    pltpu.sync_copy(out_v, out_hbm.at[pl.ds(base, B_PER_W)])


@jax.jit
def kernel(users, items, user_factors, item_factors, user_biases, item_biases):
    ub_flat = user_biases[:TAIL0].reshape(TAIL0)
    ib_flat = item_biases[:TAIL0].reshape(TAIL0)
    ub_tail = user_biases[TAIL0:].reshape(TAIL_N)
    ib_tail = item_biases[TAIL0:].reshape(TAIL_N)

    mesh = plsc.VectorSubcoreMesh(core_axis_name="c", subcore_axis_name="s")
    run_dots = functools.partial(
        pl.kernel,
        out_type=jax.ShapeDtypeStruct((BATCH,), jnp.float32),
        mesh=mesh,
        compiler_params=pltpu.CompilerParams(needs_layout_passes=False),
        scratch_types=[
            pltpu.VMEM((B_PER_W,), jnp.int32),
            pltpu.VMEM((B_PER_W,), jnp.int32),
            pltpu.VMEM((DCHUNK, N_FACTORS), jnp.float32),
            pltpu.VMEM((DCHUNK, N_FACTORS), jnp.float32),
            pltpu.VMEM((DCHUNK, N_FACTORS), jnp.float32),
            pltpu.VMEM((DCHUNK, N_FACTORS), jnp.float32),
            pltpu.VMEM((DCHUNK, N_FACTORS), jnp.float32),
            pltpu.VMEM((DCHUNK, N_FACTORS), jnp.float32),
            pltpu.VMEM((DCHUNK, N_FACTORS), jnp.float32),
            pltpu.VMEM((DCHUNK, N_FACTORS), jnp.float32),
            pltpu.VMEM((DCHUNK, 16), jnp.float32),
            pltpu.VMEM((B_PER_W,), jnp.float32),
            pltpu.SemaphoreType.DMA,
            pltpu.SemaphoreType.DMA,
            pltpu.SemaphoreType.DMA,
            pltpu.SemaphoreType.DMA,
        ],
    )(_dots_body)
    dots = run_dots(users, items, user_factors, item_factors)

    run_bias = functools.partial(
        pl.kernel,
        out_type=jax.ShapeDtypeStruct((BATCH,), jnp.float32),
        mesh=mesh,
        compiler_params=pltpu.CompilerParams(needs_layout_passes=False),
        scratch_types=[
            pltpu.VMEM((B_PER_W,), jnp.int32),
            pltpu.VMEM((B_PER_W,), jnp.int32),
            pltpu.VMEM((B_PER_W,), jnp.int32),
            pltpu.VMEM((B_PER_W,), jnp.int32),
            pltpu.VMEM((B_PER_W,), jnp.float32),
            pltpu.VMEM((B_PER_W,), jnp.float32),
            pltpu.VMEM((TAIL_N,), jnp.float32),
            pltpu.VMEM((TAIL_N,), jnp.float32),
            pltpu.VMEM((B_PER_W,), jnp.float32),
            pltpu.VMEM((B_PER_W,), jnp.float32),
            pltpu.SemaphoreType.DMA,
        ],
    )(_bias_body)
    return run_bias(users, items, dots, ub_flat, ib_flat, ub_tail, ib_tail)


# speedup vs baseline: 1.2712x; 1.0335x over previous
import functools

import jax
import jax.numpy as jnp
from jax import lax
from jax.experimental import pallas as pl
from jax.experimental.pallas import tpu as pltpu
from jax.experimental.pallas import tpu_sc as plsc

N_USERS = 1000000
N_ITEMS = 1000000
N_FACTORS = 128
BATCH = 16384

NC = 2
NS = 16
NW = NC * NS
B_PER_W = BATCH // NW
CHUNK = 128
NCHUNK = B_PER_W // CHUNK
TAIL0 = 999424
TAIL_N = N_USERS - TAIL0
DCHUNK = 64
DNCHUNK = B_PER_W // DCHUNK


def _shuf(x, idx):
    return lax.gather(
        x, idx[:, None],
        dimension_numbers=lax.GatherDimensionNumbers(
            offset_dims=(), collapsed_slice_dims=(0,), start_index_map=(0,)),
        slice_sizes=(1,),
        mode=lax.GatherScatterMode.PROMISE_IN_BOUNDS,
    )


def _dots_body(users_hbm, items_hbm, uf_hbm, if_hbm, out_hbm,
               u_idx, i_idx, uf_a, if_a, uf_b, if_b, uf_c, if_c, uf_d, if_d,
               p_dot, out_v, sem_a, sem_b, sem_c, sem_d):
    wid = lax.axis_index("s") * NC + lax.axis_index("c")
    base = wid * B_PER_W
    pltpu.sync_copy(users_hbm.at[pl.ds(base, B_PER_W)], u_idx)
    pltpu.sync_copy(items_hbm.at[pl.ds(base, B_PER_W)], i_idx)
    lane = lax.iota(jnp.int32, 16)

    bufs = [(uf_a, if_a), (uf_b, if_b), (uf_c, if_c), (uf_d, if_d)]
    sems = [sem_a, sem_b, sem_c, sem_d]
    NBUF = 4

    def fire(c):
        sl = pl.ds(c * DCHUNK, DCHUNK)
        uf_r, if_r = bufs[c % NBUF]
        sem = sems[c % NBUF]
        return [
            pltpu.async_copy(uf_hbm.at[u_idx.at[sl]], uf_r, sem),
            pltpu.async_copy(if_hbm.at[i_idx.at[sl]], if_r, sem),
        ]

    pending = {c: fire(c) for c in range(min(NBUF - 1, DNCHUNK))}
    for c in range(DNCHUNK):
        if c + NBUF - 1 < DNCHUNK:
            pending[c + NBUF - 1] = fire(c + NBUF - 1)
        for cp in pending.pop(c):
            cp.wait()
        uf_rows, if_rows = bufs[c % NBUF]

        @plsc.parallel_loop(0, DCHUNK, unroll=4)
        def rowbody(r):
            t0 = uf_rows[r, pl.ds(0, 16)] * if_rows[r, pl.ds(0, 16)]
            t1 = uf_rows[r, pl.ds(16, 16)] * if_rows[r, pl.ds(16, 16)]
            t2 = uf_rows[r, pl.ds(32, 16)] * if_rows[r, pl.ds(32, 16)]
            t3 = uf_rows[r, pl.ds(48, 16)] * if_rows[r, pl.ds(48, 16)]
            t4 = uf_rows[r, pl.ds(64, 16)] * if_rows[r, pl.ds(64, 16)]
            t5 = uf_rows[r, pl.ds(80, 16)] * if_rows[r, pl.ds(80, 16)]
            t6 = uf_rows[r, pl.ds(96, 16)] * if_rows[r, pl.ds(96, 16)]
            t7 = uf_rows[r, pl.ds(112, 16)] * if_rows[r, pl.ds(112, 16)]
            s = ((t0 + t1) + (t2 + t3)) + ((t4 + t5) + (t6 + t7))
            for k in (8, 4, 2, 1):
                s = s + _shuf(s, lane ^ k)
            p_dot[r, pl.ds(0, 16)] = s

        def gout(g, _):
            off = c * DCHUNK + g * 16
            d16 = plsc.load_gather(p_dot, [g * 16 + lane, lane])
            out_v[pl.ds(off, 16)] = d16
            return 0

        lax.fori_loop(0, DCHUNK // 16, gout, 0)
    pltpu.sync_copy(out_v, out_hbm.at[pl.ds(base, B_PER_W)])


def _bias_body(users_hbm, items_hbm, dots_hbm, ub_hbm, ib_hbm,
               ubt_hbm, ibt_hbm, out_hbm,
               u_idx, i_idx, cu_idx, ci_idx, ub_v, ib_v, ub_tl, ib_tl,
               dots_v, out_v, sem):
    wid = lax.axis_index("s") * NC + lax.axis_index("c")
    base = wid * B_PER_W
    pltpu.sync_copy(users_hbm.at[pl.ds(base, B_PER_W)], u_idx)
    pltpu.sync_copy(items_hbm.at[pl.ds(base, B_PER_W)], i_idx)
    lane = lax.iota(jnp.int32, 16)

    def crow(g, _):
        gsl = pl.ds(g * 16, 16)
        cu_idx[gsl] = jnp.minimum(u_idx[gsl], TAIL0 - 1)
        ci_idx[gsl] = jnp.minimum(i_idx[gsl], TAIL0 - 1)
        return 0

    lax.fori_loop(0, B_PER_W // 16, crow, 0)

    cps = []
    for c in range(NCHUNK):
        sl = pl.ds(c * CHUNK, CHUNK)
        cps.append(pltpu.async_copy(ub_hbm.at[cu_idx.at[sl]], ub_v.at[sl], sem))
        cps.append(pltpu.async_copy(ib_hbm.at[ci_idx.at[sl]], ib_v.at[sl], sem))
    pltpu.sync_copy(dots_hbm.at[pl.ds(base, B_PER_W)], dots_v)
    pltpu.sync_copy(ubt_hbm, ub_tl)
    pltpu.sync_copy(ibt_hbm, ib_tl)
    for cp in cps:
        cp.wait()

    def gout(g, _):
        gsl = pl.ds(g * 16, 16)
        u16 = u_idx[gsl]
        i16 = i_idx[gsl]
        ub16 = ub_v[gsl]
        ib16 = ib_v[gsl]
        ubt16 = plsc.load_gather(
            ub_tl, [jnp.maximum(jnp.minimum(u16 - TAIL0, TAIL_N - 1), 0)])
        ibt16 = plsc.load_gather(
            ib_tl, [jnp.maximum(jnp.minimum(i16 - TAIL0, TAIL_N - 1), 0)])
        ub16 = jnp.where(u16 >= TAIL0, ubt16, ub16)
        ib16 = jnp.where(i16 >= TAIL0, ibt16, ib16)
        out_v[gsl] = dots_v[gsl] + ub16 + ib16
        return 0

    lax.fori_loop(0, B_PER_W // 16, gout, 0)
    pltpu.sync_copy(out_v, out_hbm.at[pl.ds(base, B_PER_W)])


@jax.jit
def kernel(users, items, user_factors, item_factors, user_biases, item_biases):
    ub_flat = user_biases[:TAIL0].reshape(TAIL0)
    ib_flat = item_biases[:TAIL0].reshape(TAIL0)
    ub_tail = user_biases[TAIL0:].reshape(TAIL_N)
    ib_tail = item_biases[TAIL0:].reshape(TAIL_N)

    mesh = plsc.VectorSubcoreMesh(core_axis_name="c", subcore_axis_name="s")
    run_dots = functools.partial(
        pl.kernel,
        out_type=jax.ShapeDtypeStruct((BATCH,), jnp.float32),
        mesh=mesh,
        compiler_params=pltpu.CompilerParams(needs_layout_passes=False),
        scratch_types=[
            pltpu.VMEM((B_PER_W,), jnp.int32),
            pltpu.VMEM((B_PER_W,), jnp.int32),
            pltpu.VMEM((DCHUNK, N_FACTORS), jnp.float32),
            pltpu.VMEM((DCHUNK, N_FACTORS), jnp.float32),
            pltpu.VMEM((DCHUNK, N_FACTORS), jnp.float32),
            pltpu.VMEM((DCHUNK, N_FACTORS), jnp.float32),
            pltpu.VMEM((DCHUNK, N_FACTORS), jnp.float32),
            pltpu.VMEM((DCHUNK, N_FACTORS), jnp.float32),
            pltpu.VMEM((DCHUNK, N_FACTORS), jnp.float32),
            pltpu.VMEM((DCHUNK, N_FACTORS), jnp.float32),
            pltpu.VMEM((DCHUNK, 16), jnp.float32),
            pltpu.VMEM((B_PER_W,), jnp.float32),
            pltpu.SemaphoreType.DMA,
            pltpu.SemaphoreType.DMA,
            pltpu.SemaphoreType.DMA,
            pltpu.SemaphoreType.DMA,
        ],
    )(_dots_body)
    dots = run_dots(users, items, user_factors, item_factors)

    run_bias = functools.partial(
        pl.kernel,
        out_type=jax.ShapeDtypeStruct((BATCH,), jnp.float32),
        mesh=mesh,
        compiler_params=pltpu.CompilerParams(needs_layout_passes=False),
        scratch_types=[
            pltpu.VMEM((B_PER_W,), jnp.int32),
            pltpu.VMEM((B_PER_W,), jnp.int32),
            pltpu.VMEM((B_PER_W,), jnp.int32),
            pltpu.VMEM((B_PER_W,), jnp.int32),
            pltpu.VMEM((B_PER_W,), jnp.float32),
            pltpu.VMEM((B_PER_W,), jnp.float32),
            pltpu.VMEM((TAIL_N,), jnp.float32),
            pltpu.VMEM((TAIL_N,), jnp.float32),
            pltpu.VMEM((B_PER_W,), jnp.float32),
            pltpu.VMEM((B_PER_W,), jnp.float32),
            pltpu.SemaphoreType.DMA,
        ],
    )(_bias_body)
    return run_bias(users, items, dots, ub_flat, ib_flat, ub_tail, ib_tail)
